# Initial kernel scaffold; baseline (speedup 1.0000x reference)
import functools

import jax
import jax.numpy as jnp
from jax import lax
from jax.experimental import pallas as pl
from jax.experimental.pallas import tpu as pltpu
from jax.experimental.pallas import tpu_sc as plsc

NC, NS = 2, 16
NW = NC * NS
CHUNK = 128


def _emb_kernel(n_chunks_per_w, idx_hbm, table_hbm, out_hbm,
                idx_v, rows_v, gsem, osem):
    wid = lax.axis_index("s") * NC + lax.axis_index("c")
    chunk0 = wid * n_chunks_per_w

    pltpu.sync_copy(idx_hbm.at[pl.ds(chunk0, n_chunks_per_w)], idx_v)

    def gather(j, buf):
        return pltpu.async_copy(
            table_hbm.at[idx_v.at[j]], rows_v.at[buf], gsem.at[buf])

    def put(j, buf):
        return pltpu.async_copy(
            rows_v.at[buf], out_hbm.at[pl.ds((chunk0 + j) * CHUNK, CHUNK)],
            osem.at[buf])

    gather(0, 0)
    gather(1, 1)

    def body(j, _):
        buf = lax.rem(j, 2)
        gather(j, buf).wait()
        put(j, buf)
        nxt = j + 2

        @pl.when(nxt < n_chunks_per_w)
        def _():
            put(j, buf).wait()
            gather(nxt, buf)

        @pl.when(nxt >= n_chunks_per_w)
        def _():
            put(j, buf).wait()

        return 0

    lax.fori_loop(0, n_chunks_per_w, body, 0)


@jax.jit
def kernel(indices, weight):
    B0, B1 = indices.shape
    D = weight.shape[1]
    B = B0 * B1
    n_chunks = B // CHUNK
    n_chunks_per_w = n_chunks // NW

    idx2d = indices.reshape(n_chunks, CHUNK).astype(jnp.int32)
    mesh = plsc.VectorSubcoreMesh(core_axis_name="c", subcore_axis_name="s")

    out = pl.kernel(
        functools.partial(_emb_kernel, n_chunks_per_w),
        out_type=jax.ShapeDtypeStruct((B, D), jnp.float32),
        mesh=mesh,
        scratch_types=[
            pltpu.VMEM((n_chunks_per_w, CHUNK), jnp.int32),
            pltpu.VMEM((2, CHUNK, D), jnp.float32),
            pltpu.SemaphoreType.DMA((2,)),
            pltpu.SemaphoreType.DMA((2,)),
        ],
    )(idx2d, weight)
    return out.reshape(B0, B1, D)


# speedup vs baseline: 4.5339x; 4.5339x over previous
import functools

import jax
import jax.numpy as jnp
from jax import lax
from jax.experimental import pallas as pl
from jax.experimental.pallas import tpu as pltpu
from jax.experimental.pallas import tpu_sc as plsc

NC, NS = 2, 16
NW = NC * NS
CHUNK = 128


def _emb_kernel(n_chunks_per_w, idx_hbm, table_hbm, out_hbm,
                idx_v, rows_v, gsem, osem):
    wid = lax.axis_index("s") * NC + lax.axis_index("c")
    chunk0 = wid * n_chunks_per_w

    pltpu.sync_copy(idx_hbm.at[wid], idx_v)

    def gather(j, buf):
        return pltpu.make_async_copy(
            table_hbm.at[idx_v.at[j]], rows_v.at[buf], gsem.at[buf])

    def put(j, buf):
        return pltpu.make_async_copy(
            rows_v.at[buf], out_hbm.at[pl.ds((chunk0 + j) * CHUNK, CHUNK)],
            osem.at[buf])

    gather(0, 0).start()
    gather(1, 1).start()

    def body(j, _):
        buf = lax.rem(j, 2)
        gather(j, buf).wait()
        put(j, buf).start()
        put(j, buf).wait()
        nxt = j + 2

        @pl.when(nxt < n_chunks_per_w)
        def _():
            gather(nxt, buf).start()

        return 0

    lax.fori_loop(0, n_chunks_per_w, body, 0)


@jax.jit
def kernel(indices, weight):
    B0, B1 = indices.shape
    D = weight.shape[1]
    B = B0 * B1
    n_chunks = B // CHUNK
    n_chunks_per_w = n_chunks // NW

    idx3d = indices.reshape(NW, n_chunks_per_w, CHUNK).astype(jnp.int32)
    mesh = plsc.VectorSubcoreMesh(core_axis_name="c", subcore_axis_name="s")

    out = pl.kernel(
        functools.partial(_emb_kernel, n_chunks_per_w),
        out_type=jax.ShapeDtypeStruct((B, D), jnp.float32),
        mesh=mesh,
        compiler_params=pltpu.CompilerParams(use_tc_tiling_on_sc=False),
        scratch_types=[
            pltpu.VMEM((n_chunks_per_w, CHUNK), jnp.int32),
            pltpu.VMEM((2, CHUNK, D), jnp.float32),
            pltpu.SemaphoreType.DMA((2,)),
            pltpu.SemaphoreType.DMA((2,)),
        ],
    )(idx3d, weight)
    return out.reshape(B0, B1, D)


# speedup vs baseline: 4.6641x; 1.0287x over previous
import functools

import jax
import jax.numpy as jnp
from jax import lax
from jax.experimental import pallas as pl
from jax.experimental.pallas import tpu as pltpu
from jax.experimental.pallas import tpu_sc as plsc

NC, NS = 2, 16
NW = NC * NS
CHUNK = 128


GROUP = 5


def _emb_kernel(n_groups_per_w, idx_hbm, table_hbm, out_hbm,
                idx_v, rows_v, gsem, osem):
    wid = lax.axis_index("s") * NC + lax.axis_index("c")
    group0 = wid * n_groups_per_w

    pltpu.sync_copy(idx_hbm.at[wid], idx_v)

    def gather(g, k, buf):
        return pltpu.make_async_copy(
            table_hbm.at[idx_v.at[g * GROUP + k]],
            rows_v.at[buf, pl.ds(k * CHUNK, CHUNK)],
            gsem.at[buf])

    def put(g, buf):
        return pltpu.make_async_copy(
            rows_v.at[buf],
            out_hbm.at[pl.ds((group0 + g) * (GROUP * CHUNK), GROUP * CHUNK)],
            osem.at[buf])

    for k in range(GROUP):
        gather(0, k, 0).start()
    for k in range(GROUP):
        gather(1, k, 1).start()

    def body(g, _):
        buf = lax.rem(g, 2)
        for k in range(GROUP):
            gather(g, k, buf).wait()
        put(g, buf).start()
        put(g, buf).wait()
        nxt = g + 2

        @pl.when(nxt < n_groups_per_w)
        def _():
            for k in range(GROUP):
                gather(nxt, k, buf).start()

        return 0

    lax.fori_loop(0, n_groups_per_w, body, 0)


@jax.jit
def kernel(indices, weight):
    B0, B1 = indices.shape
    D = weight.shape[1]
    B = B0 * B1
    n_chunks = B // CHUNK
    n_chunks_per_w = n_chunks // NW
    n_groups_per_w = n_chunks_per_w // GROUP

    idx3d = indices.reshape(NW, n_chunks_per_w, CHUNK).astype(jnp.int32)
    mesh = plsc.VectorSubcoreMesh(core_axis_name="c", subcore_axis_name="s")

    out = pl.kernel(
        functools.partial(_emb_kernel, n_groups_per_w),
        out_type=jax.ShapeDtypeStruct((B, D), jnp.float32),
        mesh=mesh,
        compiler_params=pltpu.CompilerParams(use_tc_tiling_on_sc=False),
        scratch_types=[
            pltpu.VMEM((n_chunks_per_w, CHUNK), jnp.int32),
            pltpu.VMEM((2, GROUP * CHUNK, D), jnp.float32),
            pltpu.SemaphoreType.DMA((2,)),
            pltpu.SemaphoreType.DMA((2,)),
        ],
    )(idx3d, weight)
    return out.reshape(B0, B1, D)
